# baseline (speedup 1.0000x reference)
import functools

import jax
import jax.numpy as jnp
from jax import lax
from jax.experimental import pallas as pl
from jax.experimental.pallas import tpu as pltpu
from jax.experimental.pallas import tpu_sc as plsc

V = 100000
D = 128
B = 4096
F = 26

N = B * F
NC = 2
NS = 16
NW = NC * NS
NSPLIT = 2
BS = B // NSPLIT
BPW = BS // NW
PER_W = BPW * F
CHB = 4
CH = CHB * F
NCHUNK = BPW // CHB
L = 16
RB = 8


_GATHER_1D = lax.GatherDimensionNumbers(
    offset_dims=(), collapsed_slice_dims=(0,), start_index_map=(0,))


def _splat_lane(vec, lane):
    idx = jnp.full((L, 1), lane, jnp.int32)
    return lax.gather(vec, idx, _GATHER_1D, slice_sizes=(1,),
                      mode=lax.GatherScatterMode.PROMISE_IN_BOUNDS)


def _elu_rows(ebuf, bbuf, obuf, scl_v, slot, row0):
    @pl.loop(0, CH // RB)
    def _blk(blk):
        blk0 = blk * RB
        sblk = scl_v[pl.ds(pl.multiple_of(row0 + blk0, RB), L)]
        for r8 in range(RB):
            sv = _splat_lane(sblk, r8)
            r = blk0 + r8
            for e in range(D // L):
                col = pl.ds(e * L, L)
                x = ebuf[slot, r, col] * sv + bbuf[slot, r, col]
                obuf[slot, r, col] = jnp.maximum(
                    x, lax.exp(jnp.minimum(x, 0.0)) - 1.0)


@functools.partial(
    pl.kernel,
    out_type=jax.ShapeDtypeStruct((BS, F, D), jnp.float32),
    mesh=plsc.VectorSubcoreMesh(core_axis_name="c", subcore_axis_name="s"),
    scratch_types=[
        pltpu.VMEM((PER_W,), jnp.int32),
        pltpu.VMEM((PER_W + L,), jnp.float32),
        pltpu.VMEM((2, CH, D), jnp.float32),
        pltpu.VMEM((2, CH, D), jnp.float32),
        pltpu.VMEM((2, CH, D), jnp.float32),
        pltpu.SemaphoreType.DMA,
        pltpu.SemaphoreType.DMA,
        pltpu.SemaphoreType.DMA,
        pltpu.SemaphoreType.DMA,
        pltpu.SemaphoreType.DMA,
        pltpu.SemaphoreType.DMA,
    ],
)
def _sc_embed(tok_hbm, scl_hbm, emb_hbm, bias_hbm, out_hbm,
              idx_v, scl_v, ebuf, bbuf, obuf,
              se0, se1, sb0, sb1, so0, so1):
    sems_e = (se0, se1)
    sems_b = (sb0, sb1)
    sems_o = (so0, so1)
    wid = lax.axis_index("s") * NC + lax.axis_index("c")
    base = pl.multiple_of(wid * PER_W, PER_W)
    batch0 = pl.multiple_of(wid * BPW, BPW)

    pltpu.sync_copy(tok_hbm.at[pl.ds(base, PER_W)], idx_v)
    pltpu.sync_copy(scl_hbm.at[pl.ds(base, PER_W)], scl_v.at[pl.ds(0, PER_W)])

    def store(g, slot):
        for i in range(CHB):
            pltpu.async_copy(obuf.at[slot].at[pl.ds(i * F, F)],
                             out_hbm.at[batch0 + g * CHB + i], sems_o[slot])

    def drain_store_sem(g, slot):
        for i in range(CHB):
            pltpu.make_async_copy(obuf.at[slot].at[pl.ds(i * F, F)],
                                  out_hbm.at[batch0 + g * CHB + i],
                                  sems_o[slot]).wait()

    def issue(g, slot):
        off = pl.multiple_of(g * CH, RB)
        idx = idx_v.at[pl.ds(off, CH)]
        pltpu.async_copy(emb_hbm.at[idx], ebuf.at[slot], sems_e[slot])
        pltpu.async_copy(bias_hbm.at[idx], bbuf.at[slot], sems_b[slot])

    def wait_gathers(slot):
        idx = idx_v.at[pl.ds(0, CH)]
        pltpu.make_async_copy(emb_hbm.at[idx], ebuf.at[slot], sems_e[slot]).wait()
        pltpu.make_async_copy(bias_hbm.at[idx], bbuf.at[slot], sems_b[slot]).wait()

    def finish(g, slot, drain_store):
        wait_gathers(slot)
        if drain_store:
            drain_store_sem(g, slot)
        _elu_rows(ebuf, bbuf, obuf, scl_v, slot, g * CH)
        store(g, slot)

    for slot in range(2):
        issue(slot, slot)
    for slot in range(2):
        finish(slot, slot, drain_store=False)
        issue(slot + 2, slot)

    @pl.loop(2, NCHUNK - 2, step=2)
    def _steady(g0):
        for slot in range(2):
            finish(g0 + slot, slot, drain_store=True)
            issue(g0 + slot + 2, slot)

    for slot in range(2):
        finish(NCHUNK - 2 + slot, slot, drain_store=True)
        drain_store_sem(NCHUNK - 2 + slot, slot)


def kernel(input_tokens, inputs, embeddings, bias):
    tok = input_tokens.reshape(N).astype(jnp.int32)
    scl = inputs.reshape(N)
    ns = N // NSPLIT
    parts = [
        _sc_embed(tok[s * ns:(s + 1) * ns], scl[s * ns:(s + 1) * ns],
                  embeddings, bias)
        for s in range(NSPLIT)
    ]
    return jnp.concatenate(parts, axis=0)


# speedup vs baseline: 1.0125x; 1.0125x over previous
import functools

import jax
import jax.numpy as jnp
from jax import lax
from jax.experimental import pallas as pl
from jax.experimental.pallas import tpu as pltpu
from jax.experimental.pallas import tpu_sc as plsc

V = 100000
D = 128
B = 4096
F = 26

N = B * F
NC = 2
NS = 16
NW = NC * NS
NSPLIT = 2
BS = B // NSPLIT
BPW = BS // NW
PER_W = BPW * F
CHB = 4
CH = CHB * F
NCHUNK = BPW // CHB
L = 16
RB = 8


_GATHER_1D = lax.GatherDimensionNumbers(
    offset_dims=(), collapsed_slice_dims=(0,), start_index_map=(0,))


def _splat_lane(vec, lane):
    idx = jnp.full((L, 1), lane, jnp.int32)
    return lax.gather(vec, idx, _GATHER_1D, slice_sizes=(1,),
                      mode=lax.GatherScatterMode.PROMISE_IN_BOUNDS)


def _elu_rows(ebuf, bbuf, obuf, scl_v, slot, row0):
    @pl.loop(0, CH // RB)
    def _blk(blk):
        blk0 = blk * RB
        sblk = scl_v[pl.ds(pl.multiple_of(row0 + blk0, RB), L)]
        for r8 in range(RB):
            sv = _splat_lane(sblk, r8)
            r = blk0 + r8
            for e in range(D // L):
                col = pl.ds(e * L, L)
                x = ebuf[slot, r, col] * sv + bbuf[slot, r, col]
                obuf[slot, r, col] = jnp.maximum(
                    x, lax.exp(jnp.minimum(x, 0.0)) - 1.0)


@functools.partial(
    pl.kernel,
    out_type=jax.ShapeDtypeStruct((BS, F, D), jnp.float32),
    mesh=plsc.VectorSubcoreMesh(core_axis_name="c", subcore_axis_name="s"),
    scratch_types=[
        pltpu.VMEM((PER_W,), jnp.int32),
        pltpu.VMEM((PER_W + L,), jnp.float32),
        pltpu.VMEM((2, CH, D), jnp.float32),
        pltpu.VMEM((2, CH, D), jnp.float32),
        pltpu.VMEM((2, CH, D), jnp.float32),
        pltpu.SemaphoreType.DMA,
        pltpu.SemaphoreType.DMA,
        pltpu.SemaphoreType.DMA,
        pltpu.SemaphoreType.DMA,
        pltpu.SemaphoreType.DMA,
        pltpu.SemaphoreType.DMA,
    ],
)
def _sc_embed(tok_hbm, scl_hbm, emb_hbm, bias_hbm, out_hbm,
              idx_v, scl_v, ebuf, bbuf, obuf,
              se0, se1, sb0, sb1, so0, so1):
    sems_e = (se0, se1)
    sems_b = (sb0, sb1)
    sems_o = (so0, so1)
    wid = lax.axis_index("s") * NC + lax.axis_index("c")
    base = pl.multiple_of(wid * PER_W, PER_W)
    batch0 = pl.multiple_of(wid * BPW, BPW)

    pltpu.sync_copy(tok_hbm.at[pl.ds(base, PER_W)], idx_v)
    pltpu.sync_copy(scl_hbm.at[pl.ds(base, PER_W)], scl_v.at[pl.ds(0, PER_W)])

    def store(g, slot):
        for i in range(CHB):
            pltpu.async_copy(obuf.at[slot].at[pl.ds(i * F, F)],
                             out_hbm.at[batch0 + g * CHB + i], sems_o[slot])

    def drain_store_sem(g, slot):
        for i in range(CHB):
            pltpu.make_async_copy(obuf.at[slot].at[pl.ds(i * F, F)],
                                  out_hbm.at[batch0 + g * CHB + i],
                                  sems_o[slot]).wait()

    def issue(g, slot):
        off = pl.multiple_of(g * CH, RB)
        idx = idx_v.at[pl.ds(off, CH)]
        pltpu.async_copy(emb_hbm.at[idx], ebuf.at[slot], sems_e[slot])
        pltpu.async_copy(bias_hbm.at[idx], bbuf.at[slot], sems_b[slot])

    def wait_gathers(slot):
        idx = idx_v.at[pl.ds(0, CH)]
        pltpu.make_async_copy(emb_hbm.at[idx], ebuf.at[slot], sems_e[slot]).wait()
        pltpu.make_async_copy(bias_hbm.at[idx], bbuf.at[slot], sems_b[slot]).wait()

    def finish(g, slot, drain_store):
        wait_gathers(slot)
        if drain_store:
            drain_store_sem(g, slot)
        _elu_rows(ebuf, bbuf, obuf, scl_v, slot, g * CH)
        store(g, slot)

    for slot in range(2):
        issue(slot, slot)
    for slot in range(2):
        finish(slot, slot, drain_store=False)
        issue(slot + 2, slot)

    @pl.loop(2, NCHUNK - 2, step=2)
    def _steady(g0):
        for slot in range(2):
            finish(g0 + slot, slot, drain_store=True)
            issue(g0 + slot + 2, slot)

    for slot in range(2):
        finish(NCHUNK - 2 + slot, slot, drain_store=True)
        drain_store_sem(NCHUNK - 2 + slot, slot)


def kernel(input_tokens, inputs, embeddings, bias):
    tok = input_tokens.reshape(N).astype(jnp.int32)
    scl = inputs.reshape(N)
    ns = N // NSPLIT
    out = jnp.zeros((B, F, D), jnp.float32)
    for s in range(NSPLIT):
        part = _sc_embed(tok[s * ns:(s + 1) * ns], scl[s * ns:(s + 1) * ns],
                         embeddings, bias)
        out = lax.dynamic_update_slice(out, part, (s * BS, 0, 0))
    return out


# speedup vs baseline: 1.4229x; 1.4053x over previous
import functools

import jax
import jax.numpy as jnp
from jax import lax
from jax.experimental import pallas as pl
from jax.experimental.pallas import tpu as pltpu
from jax.experimental.pallas import tpu_sc as plsc

V = 100000
D = 128
B = 4096
F = 26

N = B * F
NC = 2
NS = 16
NW = NC * NS
BPW = B // NW
PER_W = BPW * F
CHB = 4
CH = CHB * F
NCHUNK = BPW // CHB
L = 16
RB = 8


_GATHER_1D = lax.GatherDimensionNumbers(
    offset_dims=(), collapsed_slice_dims=(0,), start_index_map=(0,))


def _splat_lane(vec, lane):
    idx = jnp.full((L, 1), lane, jnp.int32)
    return lax.gather(vec, idx, _GATHER_1D, slice_sizes=(1,),
                      mode=lax.GatherScatterMode.PROMISE_IN_BOUNDS)


def _elu_rows(ebuf, bbuf, obuf, scl_v, slot, row0):
    @pl.loop(0, CH // RB)
    def _blk(blk):
        blk0 = blk * RB
        sblk = scl_v[pl.ds(pl.multiple_of(row0 + blk0, RB), L)]
        for r8 in range(RB):
            sv = _splat_lane(sblk, r8)
            r = blk0 + r8
            for e in range(D // L):
                col = pl.ds(e * L, L)
                x = ebuf[slot, r, col] * sv + bbuf[slot, r, col]
                obuf[slot, r, col] = jnp.maximum(
                    x, lax.exp(jnp.minimum(x, 0.0)) - 1.0)


@functools.partial(
    pl.kernel,
    out_type=jax.ShapeDtypeStruct((B, F, D), jnp.float32),
    mesh=plsc.VectorSubcoreMesh(core_axis_name="c", subcore_axis_name="s"),
    scratch_types=[
        pltpu.VMEM((PER_W,), jnp.int32),
        pltpu.VMEM((PER_W + L,), jnp.float32),
        pltpu.VMEM((3, CH, D), jnp.float32),
        pltpu.VMEM((3, CH, D), jnp.float32),
        pltpu.VMEM((3, CH, D), jnp.float32),
        pltpu.SemaphoreType.DMA,
        pltpu.SemaphoreType.DMA,
        pltpu.SemaphoreType.DMA,
        pltpu.SemaphoreType.DMA,
        pltpu.SemaphoreType.DMA,
        pltpu.SemaphoreType.DMA,
        pltpu.SemaphoreType.DMA,
        pltpu.SemaphoreType.DMA,
        pltpu.SemaphoreType.DMA,
    ],
)
def _sc_embed(tok_hbm, scl_hbm, emb_hbm, bias_hbm, out_hbm,
              idx_v, scl_v, ebuf, bbuf, obuf,
              se0, se1, se2, sb0, sb1, sb2, so0, so1, so2):
    sems_e = (se0, se1, se2)
    sems_b = (sb0, sb1, sb2)
    sems_o = (so0, so1, so2)
    wid = lax.axis_index("s") * NC + lax.axis_index("c")
    base = pl.multiple_of(wid * PER_W, PER_W)
    batch0 = pl.multiple_of(wid * BPW, BPW)

    pltpu.sync_copy(tok_hbm.at[pl.ds(base, PER_W)], idx_v)
    pltpu.sync_copy(scl_hbm.at[pl.ds(base, PER_W)], scl_v.at[pl.ds(0, PER_W)])

    def store(g, slot):
        for i in range(CHB):
            pltpu.async_copy(obuf.at[slot].at[pl.ds(i * F, F)],
                             out_hbm.at[batch0 + g * CHB + i], sems_o[slot])

    def drain_store_sem(g, slot):
        for i in range(CHB):
            pltpu.make_async_copy(obuf.at[slot].at[pl.ds(i * F, F)],
                                  out_hbm.at[batch0 + g * CHB + i],
                                  sems_o[slot]).wait()

    def issue(g, slot):
        off = pl.multiple_of(g * CH, RB)
        idx = idx_v.at[pl.ds(off, CH)]
        pltpu.async_copy(emb_hbm.at[idx], ebuf.at[slot], sems_e[slot])
        pltpu.async_copy(bias_hbm.at[idx], bbuf.at[slot], sems_b[slot])

    def wait_gathers(slot):
        idx = idx_v.at[pl.ds(0, CH)]
        pltpu.make_async_copy(emb_hbm.at[idx], ebuf.at[slot], sems_e[slot]).wait()
        pltpu.make_async_copy(bias_hbm.at[idx], bbuf.at[slot], sems_b[slot]).wait()

    def finish(g, slot, drain_store):
        wait_gathers(slot)
        if drain_store:
            drain_store_sem(g, slot)
        _elu_rows(ebuf, bbuf, obuf, scl_v, slot, g * CH)
        store(g, slot)

    for slot in range(3):
        issue(slot, slot)
    for slot in range(3):
        finish(slot, slot, drain_store=False)
        issue(slot + 3, slot)

    @pl.loop(3, NCHUNK - 5, step=3)
    def _steady(g0):
        for slot in range(3):
            finish(g0 + slot, slot, drain_store=True)
            issue(g0 + slot + 3, slot)

    finish(27, 0, drain_store=True)
    issue(30, 0)
    finish(28, 1, drain_store=True)
    issue(31, 1)
    finish(29, 2, drain_store=True)
    finish(30, 0, drain_store=True)
    finish(31, 1, drain_store=True)
    for g, slot in ((30, 0), (31, 1)):
        drain_store_sem(g, slot)


def kernel(input_tokens, inputs, embeddings, bias):
    tok = input_tokens.reshape(N).astype(jnp.int32)
    scl = inputs.reshape(N)
    return _sc_embed(tok, scl, embeddings, bias)


# speedup vs baseline: 1.4314x; 1.0059x over previous
import functools

import jax
import jax.numpy as jnp
from jax import lax
from jax.experimental import pallas as pl
from jax.experimental.pallas import tpu as pltpu
from jax.experimental.pallas import tpu_sc as plsc

V = 100000
D = 128
B = 4096
F = 26

N = B * F
NC = 2
NS = 16
NW = NC * NS
BPW = B // NW
PER_W = BPW * F
CHB = 4
CH = CHB * F
NCHUNK = BPW // CHB
L = 16
RB = 8


_GATHER_1D = lax.GatherDimensionNumbers(
    offset_dims=(), collapsed_slice_dims=(0,), start_index_map=(0,))


def _splat_lane(vec, lane):
    idx = jnp.full((L, 1), lane, jnp.int32)
    return lax.gather(vec, idx, _GATHER_1D, slice_sizes=(1,),
                      mode=lax.GatherScatterMode.PROMISE_IN_BOUNDS)


def _elu_rows(ebuf, bbuf, obuf, scl_v, slot, row0):
    @pl.loop(0, CH // RB)
    def _blk(blk):
        blk0 = blk * RB
        sblk = scl_v[pl.ds(pl.multiple_of(row0 + blk0, RB), L)]
        for r8 in range(RB):
            sv = _splat_lane(sblk, r8)
            r = blk0 + r8
            for e in range(D // L):
                col = pl.ds(e * L, L)
                x = ebuf[slot, r, col] * sv + bbuf[slot, r, col]
                obuf[slot, r, col] = jnp.maximum(
                    x, lax.exp(jnp.minimum(x, 0.0)) - 1.0)


@functools.partial(
    pl.kernel,
    out_type=jax.ShapeDtypeStruct((B, F, D), jnp.float32),
    mesh=plsc.VectorSubcoreMesh(core_axis_name="c", subcore_axis_name="s"),
    scratch_types=[
        pltpu.VMEM((PER_W,), jnp.int32),
        pltpu.VMEM((PER_W + L,), jnp.float32),
        pltpu.VMEM((2, CH, D), jnp.float32),
        pltpu.VMEM((2, CH, D), jnp.float32),
        pltpu.VMEM((2, CH, D), jnp.float32),
        pltpu.SemaphoreType.DMA,
        pltpu.SemaphoreType.DMA,
        pltpu.SemaphoreType.DMA,
        pltpu.SemaphoreType.DMA,
        pltpu.SemaphoreType.DMA,
        pltpu.SemaphoreType.DMA,
    ],
)
def _sc_embed(tok_hbm, scl_hbm, emb_hbm, bias_hbm, out_hbm,
              idx_v, scl_v, ebuf, bbuf, obuf,
              se0, se1, sb0, sb1, so0, so1):
    sems_e = (se0, se1)
    sems_b = (sb0, sb1)
    sems_o = (so0, so1)
    wid = lax.axis_index("s") * NC + lax.axis_index("c")
    base = pl.multiple_of(wid * PER_W, PER_W)
    batch0 = pl.multiple_of(wid * BPW, BPW)

    pltpu.sync_copy(tok_hbm.at[pl.ds(base, PER_W)], idx_v)
    pltpu.sync_copy(scl_hbm.at[pl.ds(base, PER_W)], scl_v.at[pl.ds(0, PER_W)])

    def store(g, slot):
        for i in range(CHB):
            pltpu.async_copy(obuf.at[slot].at[pl.ds(i * F, F)],
                             out_hbm.at[batch0 + g * CHB + i], sems_o[slot])

    def drain_store_sem(g, slot):
        for i in range(CHB):
            pltpu.make_async_copy(obuf.at[slot].at[pl.ds(i * F, F)],
                                  out_hbm.at[batch0 + g * CHB + i],
                                  sems_o[slot]).wait()

    def issue(g, slot):
        off = pl.multiple_of(g * CH, RB)
        idx = idx_v.at[pl.ds(off, CH)]
        pltpu.async_copy(emb_hbm.at[idx], ebuf.at[slot], sems_e[slot])
        pltpu.async_copy(bias_hbm.at[idx], bbuf.at[slot], sems_b[slot])

    def wait_gathers(slot):
        idx = idx_v.at[pl.ds(0, CH)]
        pltpu.make_async_copy(emb_hbm.at[idx], ebuf.at[slot], sems_e[slot]).wait()
        pltpu.make_async_copy(bias_hbm.at[idx], bbuf.at[slot], sems_b[slot]).wait()

    def finish(g, slot, drain_store):
        wait_gathers(slot)
        if drain_store:
            drain_store_sem(g, slot)
        _elu_rows(ebuf, bbuf, obuf, scl_v, slot, g * CH)
        store(g, slot)

    for slot in range(2):
        issue(slot, slot)
    for slot in range(2):
        finish(slot, slot, drain_store=False)
        issue(slot + 2, slot)

    @pl.loop(2, NCHUNK - 2, step=2)
    def _steady(g0):
        for slot in range(2):
            finish(g0 + slot, slot, drain_store=True)
            issue(g0 + slot + 2, slot)

    for slot in range(2):
        finish(NCHUNK - 2 + slot, slot, drain_store=True)
        drain_store_sem(NCHUNK - 2 + slot, slot)


def kernel(input_tokens, inputs, embeddings, bias):
    tok = input_tokens.reshape(N).astype(jnp.int32)
    scl = inputs.reshape(N)
    return _sc_embed(tok, scl, embeddings, bias)
